# Initial kernel scaffold; baseline (speedup 1.0000x reference)
import jax
import jax.numpy as jnp
from jax.experimental import pallas as pl


def kernel(feat, segment_ids, W1, b1, W2, b2):
    raise NotImplementedError("write your pallas kernel here")


# speedup vs baseline: 2.7139x; 2.7139x over previous
import functools

import jax
import jax.numpy as jnp
from jax.experimental import pallas as pl
from jax.experimental.pallas import tpu as pltpu

NODE_NUM = 27
N_PER_TYPE = 8192
IN_DIM = 300
HIDDEN_DIM = 128
OUT_DIM = 64
BATCH = 16

CHUNK = 1024
NCHUNK = N_PER_TYPE // CHUNK


def _tc_body(seg_ref, feat_ref, w1_ref, b1_ref, w2_ref, b2_ref, out_ref, acc_ref):
    c = pl.program_id(1)
    seg_chunk = seg_ref[0, 0, pl.ds(c * CHUNK, CHUNK)]
    iota = jax.lax.broadcasted_iota(jnp.int32, (BATCH, CHUNK), 0)
    onehot = (seg_chunk[None, :] == iota).astype(jnp.bfloat16)
    partial = jnp.dot(onehot, feat_ref[0].astype(jnp.bfloat16),
                      preferred_element_type=jnp.float32)

    @pl.when(c == 0)
    def _():
        acc_ref[...] = partial

    @pl.when(c > 0)
    def _():
        acc_ref[...] += partial

    @pl.when(c == NCHUNK - 1)
    def _():
        seg_row = seg_ref[0, 0, :]
        iota_b = jax.lax.broadcasted_iota(jnp.int32, (BATCH, N_PER_TYPE), 0)
        counts = jnp.sum((seg_row[None, :] == iota_b).astype(jnp.float32),
                         axis=1)
        mean = jnp.where(counts[:, None] > 0,
                         acc_ref[...] / jnp.maximum(counts, 1.0)[:, None],
                         0.0)
        h = jnp.dot(mean.astype(jnp.bfloat16), w1_ref[0].astype(jnp.bfloat16),
                    preferred_element_type=jnp.float32) + b1_ref[0]
        h = jnp.maximum(h, 0.0)
        out = jnp.dot(h.astype(jnp.bfloat16), w2_ref[0].astype(jnp.bfloat16),
                      preferred_element_type=jnp.float32) + b2_ref[0]
        out_ref[0] = out


@jax.jit
def kernel(feat, segment_ids, W1, b1, W2, b2):
    seg3 = segment_ids.reshape(NODE_NUM, 1, N_PER_TYPE)
    b1r = b1.reshape(NODE_NUM, 1, HIDDEN_DIM)
    b2r = b2.reshape(NODE_NUM, 1, OUT_DIM)
    out = pl.pallas_call(
        _tc_body,
        grid=(NODE_NUM, NCHUNK),
        in_specs=[
            pl.BlockSpec((1, 1, N_PER_TYPE), lambda t, c: (t, 0, 0)),
            pl.BlockSpec((1, CHUNK, IN_DIM), lambda t, c: (t, c, 0)),
            pl.BlockSpec((1, IN_DIM, HIDDEN_DIM), lambda t, c: (t, 0, 0)),
            pl.BlockSpec((1, 1, HIDDEN_DIM), lambda t, c: (t, 0, 0)),
            pl.BlockSpec((1, HIDDEN_DIM, OUT_DIM), lambda t, c: (t, 0, 0)),
            pl.BlockSpec((1, 1, OUT_DIM), lambda t, c: (t, 0, 0)),
        ],
        out_specs=pl.BlockSpec((1, BATCH, OUT_DIM), lambda t, c: (t, 0, 0)),
        out_shape=jax.ShapeDtypeStruct((NODE_NUM, BATCH, OUT_DIM), jnp.float32),
        scratch_shapes=[pltpu.VMEM((BATCH, IN_DIM), jnp.float32)],
    )(seg3, feat, W1, b1r, W2, b2r)
    return jnp.transpose(out, (1, 0, 2))
